# baseline (speedup 1.0000x reference)
import functools

import jax
import jax.numpy as jnp
from jax import lax
from jax.experimental import pallas as pl
from jax.experimental.pallas import tpu as pltpu
from jax.experimental.pallas import tpu_sc as plsc

_E, _K, _D, _F, _T = 8, 2, 1024, 768, 2048
_TM = 256
_NT = (_T * _K) // _TM
_U = _NT + _E - 1


_NF = 2
_FC = _F // _NF


def _gmm_body(unit_e_ref, unit_end_ref, x_ref, w1_ref, w3_ref, w2_ref, wr_ref,
              y_ref):
    u = pl.program_id(0)
    j = pl.program_id(1)
    rows = lax.broadcasted_iota(jnp.int32, (_TM, 1), 0) + u * _TM
    valid = rows < unit_end_ref[u, 0]
    x = jnp.where(valid, x_ref[...], 0.0).astype(jnp.bfloat16)
    a = jnp.dot(x, w1_ref[0].astype(jnp.bfloat16),
                preferred_element_type=jnp.float32)
    b = jnp.dot(x, w3_ref[0].astype(jnp.bfloat16),
                preferred_element_type=jnp.float32)
    h = (a * jax.nn.sigmoid(a) * b).astype(jnp.bfloat16)
    yj = jnp.dot(h, w2_ref[0].astype(jnp.bfloat16),
                 preferred_element_type=jnp.float32) * wr_ref[...]

    @pl.when(j == 0)
    def _():
        y_ref[...] = yj

    @pl.when(j != 0)
    def _():
        y_ref[...] += yj


def _gmm(unit_e, unit_end, x_pad, w1, w3, w2, w_pad):
    grid_spec = pltpu.PrefetchScalarGridSpec(
        num_scalar_prefetch=2,
        grid=(_U, _NF),
        in_specs=[
            pl.BlockSpec((_TM, _D), lambda u, j, ue, un: (u, 0)),
            pl.BlockSpec((1, _D, _FC), lambda u, j, ue, un: (ue[u, 0], 0, j)),
            pl.BlockSpec((1, _D, _FC), lambda u, j, ue, un: (ue[u, 0], 0, j)),
            pl.BlockSpec((1, _FC, _D), lambda u, j, ue, un: (ue[u, 0], j, 0)),
            pl.BlockSpec((_TM, 1), lambda u, j, ue, un: (u, 0)),
        ],
        out_specs=pl.BlockSpec((_TM, _D), lambda u, j, ue, un: (u, 0)),
    )
    return pl.pallas_call(
        _gmm_body,
        grid_spec=grid_spec,
        out_shape=jax.ShapeDtypeStruct((_U * _TM, _D), jnp.float32),
    )(unit_e, unit_end, x_pad, w1, w3, w2, w_pad)


def _router_body(p_ref, pos_ref, w_ref, ue_ref, un_ref, cb_ref):
    p = p_ref[...]
    col = lax.broadcasted_iota(jnp.int32, (_T, _E), 1)
    m1 = jnp.max(p, axis=1, keepdims=True)
    e0 = jnp.min(jnp.where(p == m1, col, _E), axis=1, keepdims=True)
    p2 = jnp.where(col == e0, -1.0, p)
    m2 = jnp.max(p2, axis=1, keepdims=True)
    e1 = jnp.min(jnp.where(p2 == m2, col, _E), axis=1, keepdims=True)
    s = m1 + m2 + 1e-20
    w_ref[...] = jnp.concatenate([m1 / s, m2 / s], axis=1)

    oh = ((col == e0) | (col == e1)).astype(jnp.bfloat16)
    tri = (lax.broadcasted_iota(jnp.int32, (_RB, _RB), 0)
           > lax.broadcasted_iota(jnp.int32, (_RB, _RB), 1)).astype(jnp.bfloat16)
    carry = jnp.zeros((1, _E), jnp.float32)
    for i in range(_T // _RB):
        blk = oh[i * _RB:(i + 1) * _RB, :]
        cb_ref[i * _RB:(i + 1) * _RB, :] = (
            jnp.dot(tri, blk, preferred_element_type=jnp.float32) + carry)
        carry = carry + jnp.sum(blk.astype(jnp.float32), axis=0, keepdims=True)

    counts = carry
    upe256 = jnp.ceil(counts * (1.0 / _TM)) * _TM
    cb = cb_ref[...]
    rank0 = jnp.sum(jnp.where(col == e0, cb, 0.0), axis=1, keepdims=True)
    rank1 = jnp.sum(jnp.where(col == e1, cb, 0.0), axis=1, keepdims=True)
    off0 = jnp.sum(jnp.where(col < e0, upe256, 0.0), axis=1, keepdims=True)
    off1 = jnp.sum(jnp.where(col < e1, upe256, 0.0), axis=1, keepdims=True)
    pos_ref[...] = jnp.concatenate(
        [(off0 + rank0), (off1 + rank1)], axis=1).astype(jnp.int32)

    ucol = lax.broadcasted_iota(jnp.int32, (_UP, _E), 1)
    uidx = (lax.broadcasted_iota(jnp.int32, (_UP, _E), 0) * _TM
            ).astype(jnp.float32)
    cum256 = jnp.dot(jnp.broadcast_to(upe256.astype(jnp.bfloat16), (_E, _E)),
                     (lax.broadcasted_iota(jnp.int32, (_E, _E), 0)
                      <= lax.broadcasted_iota(jnp.int32, (_E, _E), 1)
                      ).astype(jnp.bfloat16),
                     preferred_element_type=jnp.float32)[0:1, :]
    ue = jnp.sum((cum256 <= uidx).astype(jnp.int32), axis=1, keepdims=True)
    ue = jnp.clip(ue, 0, _E - 1)
    uend = (jnp.sum(jnp.where(ucol < ue, upe256, 0.0), axis=1, keepdims=True)
            + jnp.sum(jnp.where(ucol == ue, counts, 0.0), axis=1,
                      keepdims=True))
    ue_ref[...] = jnp.broadcast_to(ue, (_UP, _E))
    un_ref[...] = jnp.broadcast_to(uend.astype(jnp.int32), (_UP, _E))


_RB = 128
_UP = ((_U + 7) // 8) * 8


def _router(probs):
    return pl.pallas_call(
        _router_body,
        out_shape=[
            jax.ShapeDtypeStruct((_T, _K), jnp.int32),
            jax.ShapeDtypeStruct((_T, _K), jnp.float32),
            jax.ShapeDtypeStruct((_UP, _E), jnp.int32),
            jax.ShapeDtypeStruct((_UP, _E), jnp.int32),
        ],
        scratch_shapes=[pltpu.VMEM((_T, _E), jnp.float32)],
    )(probs)


_NW = 32
_TPW = _T // _NW


def _dispatch_body(x_hbm, pos_hbm, xpad_hbm, rows_v, idx0_v, idx1_v, sem):
    wid = lax.axis_index("s") * 2 + lax.axis_index("c")
    t0 = wid * _TPW
    pltpu.sync_copy(x_hbm.at[pl.ds(t0, _TPW)], rows_v)
    pltpu.sync_copy(pos_hbm.at[pl.ds(t0, _TPW)], idx0_v)
    pltpu.sync_copy(pos_hbm.at[pl.ds(_T + t0, _TPW)], idx1_v)
    c0 = pltpu.async_copy(rows_v, xpad_hbm.at[idx0_v], sem)
    c1 = pltpu.async_copy(rows_v, xpad_hbm.at[idx1_v], sem)
    c0.wait()
    c1.wait()


_CC = 32


def _combine_body(y_hbm, pos_hbm, out_hbm, rows0_v, rows1_v, idx0_v, idx1_v,
                  sem):
    wid = lax.axis_index("s") * 2 + lax.axis_index("c")
    t0 = wid * _TPW
    pltpu.sync_copy(pos_hbm.at[pl.ds(t0, _TPW)], idx0_v)
    pltpu.sync_copy(pos_hbm.at[pl.ds(_T + t0, _TPW)], idx1_v)
    for c in range(_TPW // _CC):
        g0 = pltpu.async_copy(y_hbm.at[idx0_v.at[pl.ds(c * _CC, _CC)]],
                              rows0_v, sem)
        g1 = pltpu.async_copy(y_hbm.at[idx1_v.at[pl.ds(c * _CC, _CC)]],
                              rows1_v, sem)
        g0.wait()
        g1.wait()

        def _add_row(i, carry):
            for j in range(_D // 16):
                sl = pl.ds(j * 16, 16)
                rows0_v[i, sl] = rows0_v[i, sl] + rows1_v[i, sl]
            return carry

        lax.fori_loop(0, _CC, _add_row, 0)
        pltpu.sync_copy(rows0_v, out_hbm.at[pl.ds(t0 + c * _CC, _CC)])


def _combine(y_pad, pos_flat):
    mesh = plsc.VectorSubcoreMesh(core_axis_name="c", subcore_axis_name="s")
    return pl.kernel(
        _combine_body,
        out_type=jax.ShapeDtypeStruct((_T, _D), jnp.float32),
        mesh=mesh,
        scratch_types=[
            pltpu.VMEM((_CC, _D), jnp.float32),
            pltpu.VMEM((_CC, _D), jnp.float32),
            pltpu.VMEM((_TPW,), jnp.int32),
            pltpu.VMEM((_TPW,), jnp.int32),
            pltpu.SemaphoreType.DMA,
        ],
    )(y_pad, pos_flat)


def _dispatch(x, pos_flat):
    mesh = plsc.VectorSubcoreMesh(core_axis_name="c", subcore_axis_name="s")
    return pl.kernel(
        _dispatch_body,
        out_type=jax.ShapeDtypeStruct((_U * _TM, _D), jnp.float32),
        mesh=mesh,
        scratch_types=[
            pltpu.VMEM((_TPW, _D), jnp.float32),
            pltpu.VMEM((_TPW,), jnp.int32),
            pltpu.VMEM((_TPW,), jnp.int32),
            pltpu.SemaphoreType.DMA,
        ],
    )(x, pos_flat)


def kernel(hidden_states, gate_weight, w1, w3, w2):
    x = hidden_states

    logits = x @ gate_weight.T
    probs = jax.nn.softmax(logits, axis=-1)
    pos_pair, w_pair, unit_e, unit_end = _router(probs)

    pos_flat = pos_pair.T.reshape(-1)
    x_pad = _dispatch(x, pos_flat)

    w_pad = jnp.zeros((_U * _TM, 1), jnp.float32).at[
        pos_pair.reshape(-1), 0].set(w_pair.reshape(-1))

    y_pad = _gmm(unit_e, unit_end, x_pad, w1, w3, w2, w_pad)

    return _combine(y_pad, pos_flat)


# speedup vs baseline: 1.3020x; 1.3020x over previous
import functools

import jax
import jax.numpy as jnp
from jax import lax
from jax.experimental import pallas as pl
from jax.experimental.pallas import tpu as pltpu
from jax.experimental.pallas import tpu_sc as plsc

_E, _K, _D, _F, _T = 8, 2, 1024, 768, 2048
_TM = 512
_NT = (_T * _K) // _TM
_U = _NT + _E - 1


def _gmm_body(unit_e_ref, unit_end_ref, x_ref, w1_ref, w3_ref, w2_ref, wr_ref,
              y_ref):
    u = pl.program_id(0)
    rows = lax.broadcasted_iota(jnp.int32, (_TM, 1), 0) + u * _TM
    valid = rows < unit_end_ref[u, 0]
    x = jnp.where(valid, x_ref[...], 0.0).astype(jnp.bfloat16)
    a = jnp.dot(x, w1_ref[0].astype(jnp.bfloat16),
                preferred_element_type=jnp.float32)
    b = jnp.dot(x, w3_ref[0].astype(jnp.bfloat16),
                preferred_element_type=jnp.float32)
    h = (a * jax.nn.sigmoid(a) * b).astype(jnp.bfloat16)
    y = jnp.dot(h, w2_ref[0].astype(jnp.bfloat16),
                preferred_element_type=jnp.float32)
    y_ref[...] = y * wr_ref[...]


def _gmm(unit_e, unit_end, x_pad, w1, w3, w2, w_pad):
    grid_spec = pltpu.PrefetchScalarGridSpec(
        num_scalar_prefetch=2,
        grid=(_U,),
        in_specs=[
            pl.BlockSpec((_TM, _D), lambda u, ue, un: (u, 0)),
            pl.BlockSpec((1, _D, _F), lambda u, ue, un: (ue[u, 0], 0, 0)),
            pl.BlockSpec((1, _D, _F), lambda u, ue, un: (ue[u, 0], 0, 0)),
            pl.BlockSpec((1, _F, _D), lambda u, ue, un: (ue[u, 0], 0, 0)),
            pl.BlockSpec((_TM, 1), lambda u, ue, un: (u, 0)),
        ],
        out_specs=pl.BlockSpec((_TM, _D), lambda u, ue, un: (u, 0)),
    )
    return pl.pallas_call(
        _gmm_body,
        grid_spec=grid_spec,
        out_shape=jax.ShapeDtypeStruct((_U * _TM, _D), jnp.float32),
    )(unit_e, unit_end, x_pad, w1, w3, w2, w_pad)


def _router_body(p_ref, pos_ref, w_ref, ue_ref, un_ref, cb_ref):
    p = p_ref[...]
    col = lax.broadcasted_iota(jnp.int32, (_T, _E), 1)
    m1 = jnp.max(p, axis=1, keepdims=True)
    e0 = jnp.min(jnp.where(p == m1, col, _E), axis=1, keepdims=True)
    p2 = jnp.where(col == e0, -1.0, p)
    m2 = jnp.max(p2, axis=1, keepdims=True)
    e1 = jnp.min(jnp.where(p2 == m2, col, _E), axis=1, keepdims=True)
    s = m1 + m2 + 1e-20
    w_ref[...] = jnp.concatenate([m1 / s, m2 / s], axis=1)

    oh = ((col == e0) | (col == e1)).astype(jnp.bfloat16)
    tri = (lax.broadcasted_iota(jnp.int32, (_RB, _RB), 0)
           > lax.broadcasted_iota(jnp.int32, (_RB, _RB), 1)).astype(jnp.bfloat16)
    carry = jnp.zeros((1, _E), jnp.float32)
    for i in range(_T // _RB):
        blk = oh[i * _RB:(i + 1) * _RB, :]
        cb_ref[i * _RB:(i + 1) * _RB, :] = (
            jnp.dot(tri, blk, preferred_element_type=jnp.float32) + carry)
        carry = carry + jnp.sum(blk.astype(jnp.float32), axis=0, keepdims=True)

    counts = carry
    upe256 = jnp.ceil(counts * (1.0 / _TM)) * _TM
    cb = cb_ref[...]
    rank0 = jnp.sum(jnp.where(col == e0, cb, 0.0), axis=1, keepdims=True)
    rank1 = jnp.sum(jnp.where(col == e1, cb, 0.0), axis=1, keepdims=True)
    off0 = jnp.sum(jnp.where(col < e0, upe256, 0.0), axis=1, keepdims=True)
    off1 = jnp.sum(jnp.where(col < e1, upe256, 0.0), axis=1, keepdims=True)
    pos_ref[...] = jnp.concatenate(
        [(off0 + rank0), (off1 + rank1)], axis=1).astype(jnp.int32)

    ucol = lax.broadcasted_iota(jnp.int32, (_UP, _E), 1)
    uidx = (lax.broadcasted_iota(jnp.int32, (_UP, _E), 0) * _TM
            ).astype(jnp.float32)
    cum256 = jnp.dot(jnp.broadcast_to(upe256.astype(jnp.bfloat16), (_E, _E)),
                     (lax.broadcasted_iota(jnp.int32, (_E, _E), 0)
                      <= lax.broadcasted_iota(jnp.int32, (_E, _E), 1)
                      ).astype(jnp.bfloat16),
                     preferred_element_type=jnp.float32)[0:1, :]
    ue = jnp.sum((cum256 <= uidx).astype(jnp.int32), axis=1, keepdims=True)
    ue = jnp.clip(ue, 0, _E - 1)
    uend = (jnp.sum(jnp.where(ucol < ue, upe256, 0.0), axis=1, keepdims=True)
            + jnp.sum(jnp.where(ucol == ue, counts, 0.0), axis=1,
                      keepdims=True))
    ue_ref[...] = jnp.broadcast_to(ue, (_UP, _E))
    un_ref[...] = jnp.broadcast_to(uend.astype(jnp.int32), (_UP, _E))


_RB = 128
_UP = ((_U + 7) // 8) * 8


def _router(probs):
    return pl.pallas_call(
        _router_body,
        out_shape=[
            jax.ShapeDtypeStruct((_T, _K), jnp.int32),
            jax.ShapeDtypeStruct((_T, _K), jnp.float32),
            jax.ShapeDtypeStruct((_UP, _E), jnp.int32),
            jax.ShapeDtypeStruct((_UP, _E), jnp.int32),
        ],
        scratch_shapes=[pltpu.VMEM((_T, _E), jnp.float32)],
    )(probs)


_NW = 32
_TPW = _T // _NW


def _dispatch_body(x_hbm, pos_hbm, xpad_hbm, rows_v, idx0_v, idx1_v, sem):
    wid = lax.axis_index("s") * 2 + lax.axis_index("c")
    t0 = wid * _TPW
    pltpu.sync_copy(x_hbm.at[pl.ds(t0, _TPW)], rows_v)
    pltpu.sync_copy(pos_hbm.at[pl.ds(t0, _TPW)], idx0_v)
    pltpu.sync_copy(pos_hbm.at[pl.ds(_T + t0, _TPW)], idx1_v)
    c0 = pltpu.async_copy(rows_v, xpad_hbm.at[idx0_v], sem)
    c1 = pltpu.async_copy(rows_v, xpad_hbm.at[idx1_v], sem)
    c0.wait()
    c1.wait()


_CC = 32


def _combine_body(y_hbm, pos_hbm, out_hbm, rows0_v, rows1_v, idx0_v, idx1_v,
                  sem):
    wid = lax.axis_index("s") * 2 + lax.axis_index("c")
    t0 = wid * _TPW
    pltpu.sync_copy(pos_hbm.at[pl.ds(t0, _TPW)], idx0_v)
    pltpu.sync_copy(pos_hbm.at[pl.ds(_T + t0, _TPW)], idx1_v)
    for c in range(_TPW // _CC):
        g0 = pltpu.async_copy(y_hbm.at[idx0_v.at[pl.ds(c * _CC, _CC)]],
                              rows0_v, sem)
        g1 = pltpu.async_copy(y_hbm.at[idx1_v.at[pl.ds(c * _CC, _CC)]],
                              rows1_v, sem)
        g0.wait()
        g1.wait()

        def _add_row(i, carry):
            for j in range(_D // 16):
                sl = pl.ds(j * 16, 16)
                rows0_v[i, sl] = rows0_v[i, sl] + rows1_v[i, sl]
            return carry

        lax.fori_loop(0, _CC, _add_row, 0)
        pltpu.sync_copy(rows0_v, out_hbm.at[pl.ds(t0 + c * _CC, _CC)])


def _combine(y_pad, pos_flat):
    mesh = plsc.VectorSubcoreMesh(core_axis_name="c", subcore_axis_name="s")
    return pl.kernel(
        _combine_body,
        out_type=jax.ShapeDtypeStruct((_T, _D), jnp.float32),
        mesh=mesh,
        scratch_types=[
            pltpu.VMEM((_CC, _D), jnp.float32),
            pltpu.VMEM((_CC, _D), jnp.float32),
            pltpu.VMEM((_TPW,), jnp.int32),
            pltpu.VMEM((_TPW,), jnp.int32),
            pltpu.SemaphoreType.DMA,
        ],
    )(y_pad, pos_flat)


def _dispatch(x, pos_flat):
    mesh = plsc.VectorSubcoreMesh(core_axis_name="c", subcore_axis_name="s")
    return pl.kernel(
        _dispatch_body,
        out_type=jax.ShapeDtypeStruct((_U * _TM, _D), jnp.float32),
        mesh=mesh,
        scratch_types=[
            pltpu.VMEM((_TPW, _D), jnp.float32),
            pltpu.VMEM((_TPW,), jnp.int32),
            pltpu.VMEM((_TPW,), jnp.int32),
            pltpu.SemaphoreType.DMA,
        ],
    )(x, pos_flat)


def kernel(hidden_states, gate_weight, w1, w3, w2):
    x = hidden_states

    logits = x @ gate_weight.T
    probs = jax.nn.softmax(logits, axis=-1)
    pos_pair, w_pair, unit_e, unit_end = _router(probs)

    pos_flat = pos_pair.T.reshape(-1)
    x_pad = _dispatch(x, pos_flat)

    w_pad = jnp.zeros((_U * _TM, 1), jnp.float32).at[
        pos_pair.reshape(-1), 0].set(w_pair.reshape(-1))

    y_pad = _gmm(unit_e, unit_end, x_pad, w1, w3, w2, w_pad)

    return _combine(y_pad, pos_flat)
